# baseline (speedup 1.0000x reference)
import functools

import jax
import jax.numpy as jnp
from jax import lax
from jax.experimental import pallas as pl
from jax.experimental.pallas import tpu as pltpu
from jax.experimental.pallas import tpu_sc as plsc

N0, N1, N2 = 10000, 7500, 3000
D = 128
DA = 144
NC, NS = 2, 16
NW = NC * NS
CHUNK = 128


def _sc_aggregate(n_chunks, acc_rows, out_rows, filter_below=None):
    assert n_chunks % 2 == 0
    crows = n_chunks + 4
    zrows = acc_rows // NS
    orows = out_rows // NS

    mesh = plsc.VectorSubcoreMesh(
        core_axis_name="c", subcore_axis_name="s",
        num_cores=NC, num_subcores=NS)

    scratch = [
        pltpu.VMEM_SHARED((acc_rows, DA), jnp.float32),
        pltpu.VMEM((n_chunks + 2, CHUNK), jnp.int32),
        pltpu.VMEM((n_chunks + 2, CHUNK), jnp.int32),
        pltpu.VMEM((CHUNK, DA), jnp.float32),
        pltpu.VMEM((CHUNK, DA), jnp.float32),
        pltpu.VMEM((16, DA), jnp.float32),
        pltpu.SemaphoreType.DMA,
        pltpu.SemaphoreType.DMA,
    ]
    if filter_below is not None:
        scratch += [
            pltpu.VMEM((crows * CHUNK,), jnp.int32),
            pltpu.VMEM((crows * CHUNK,), jnp.int32),
        ]

    @functools.partial(
        pl.kernel,
        out_type=jax.ShapeDtypeStruct((NC, out_rows, DA), jnp.float32),
        mesh=mesh,
        scratch_types=scratch,
        compiler_params=pltpu.CompilerParams(use_tc_tiling_on_sc=False, needs_layout_passes=False),
    )
    def agg(table, srcs, dsts, *rest):
        if filter_below is None:
            (parts, acc, src_v, dst_v, rows0, rows1, zbuf,
             sem0, sem1) = rest
        else:
            (cinit, parts, acc, src_v, dst_v, rows0, rows1, zbuf,
             sem0, sem1, csrc, cdst) = rest
        c = lax.axis_index("c")
        s = lax.axis_index("s")
        wid = c * NS + s

        for i in range(16):
            for j in range(DA // 16):
                zbuf[i, pl.ds(j * 16, 16)] = jnp.zeros((16,), jnp.float32)

        def zero_body(k, _):
            pltpu.sync_copy(zbuf, acc.at[pl.ds(s * zrows + k * 16, 16)])
            return 0
        lax.fori_loop(0, zrows // 16, zero_body, 0)

        pltpu.sync_copy(srcs.at[wid], src_v)
        pltpu.sync_copy(dsts.at[wid], dst_v)
        plsc.subcore_barrier()

        if filter_below is None:
            gsrc, gdst = src_v, dst_v
            n_loop = n_chunks
        else:
            pltpu.sync_copy(cinit.at[0], csrc)
            pltpu.sync_copy(cinit.at[1], cdst)

            def cmp_body(j, off):
                for t in range(CHUNK // 16):
                    sv = src_v[j, pl.ds(t * 16, 16)]
                    dv = dst_v[j, pl.ds(t * 16, 16)]
                    m = dv < filter_below
                    plsc.store_compressed(csrc.at[pl.ds(off, 16)], sv, mask=m)
                    plsc.store_compressed(cdst.at[pl.ds(off, 16)], dv, mask=m)
                    off = off + plsc.all_reduce_population_count(m)[0]
                return off
            n_kept = lax.fori_loop(0, n_chunks, cmp_body,
                                   jnp.int32(0))
            gsrc, gdst = csrc, cdst
            n_loop = jnp.right_shift(n_kept + CHUNK - 1, 7)

        if filter_below is None:
            def edge_body(j, _):
                pltpu.async_copy(table.at[gsrc.at[j]], rows0, sem0).wait()
                pltpu.sync_copy(rows0, acc.at[gdst.at[j]], add=True)
                return 0
        else:
            def edge_body(j, _):
                sl = pl.ds(j * CHUNK, CHUNK)
                pltpu.async_copy(table.at[gsrc.at[sl]], rows0, sem0).wait()
                pltpu.sync_copy(rows0, acc.at[gdst.at[sl]], add=True)
                return 0
        lax.fori_loop(0, n_loop, edge_body, 0)

        plsc.subcore_barrier()
        pltpu.sync_copy(acc.at[pl.ds(s * orows, orows)],
                        parts.at[c, pl.ds(s * orows, orows)])

    return agg


def _tc_dense(parts, x_tgt, W_l, b_l, W_r, relu, aug):
    n = x_tgt.shape[0]
    out_w = DA if aug else D

    def body(p_ref, x_ref, wl_ref, bl_ref, wr_ref, o_ref):
        acc = p_ref[0] + p_ref[1]
        feat = acc[:n, :D]
        cnt = acc[:n, D:D + 1]
        mean = feat / jnp.maximum(cnt, 1.0)
        h = (jnp.dot(mean, wl_ref[...], preferred_element_type=jnp.float32)
             + bl_ref[...]
             + jnp.dot(x_ref[...], wr_ref[...],
                       preferred_element_type=jnp.float32))
        if relu:
            h = jnp.maximum(h, 0.0)
        if aug:
            o_ref[:, :D] = h
            lane = lax.broadcasted_iota(jnp.int32, (n, DA - D), 1)
            o_ref[:, D:] = jnp.where(lane == 0, 1.0, 0.0)
        else:
            o_ref[...] = h

    return pl.pallas_call(
        body,
        out_shape=jax.ShapeDtypeStruct((n, out_w), jnp.float32),
    )(parts, x_tgt, W_l, b_l, W_r)


def _pad_edges(edge_index, n_chunks, dummy, spread=72):
    e = edge_index.shape[1]
    total = NW * n_chunks * CHUNK
    src = jnp.concatenate(
        [edge_index[0].astype(jnp.int32),
         jnp.zeros((total - e,), jnp.int32)])
    dst = jnp.concatenate(
        [edge_index[1].astype(jnp.int32),
         dummy + jnp.arange(total - e, dtype=jnp.int32) % spread])
    src = src.reshape(NW, n_chunks, CHUNK)
    dst = dst.reshape(NW, n_chunks, CHUNK)
    zpad = jnp.zeros((NW, 2, CHUNK), jnp.int32)
    dpad = jnp.full((NW, 2, CHUNK), dummy, jnp.int32)
    return (jnp.concatenate([src, zpad], axis=1),
            jnp.concatenate([dst, dpad], axis=1))


def kernel(x, edge_index_0, edge_index_1, W_l0, b_l0, W_r0, W_l1, b_l1, W_r1):
    tbl0 = jnp.concatenate(
        [x[:N1], jnp.ones((N1, 1), jnp.float32),
         jnp.zeros((N1, DA - D - 1), jnp.float32)], axis=1)
    src0, dst0 = _pad_edges(edge_index_0, 60, N1)
    cinit0 = jnp.stack([jnp.zeros((64 * CHUNK,), jnp.int32),
                        N2 + jnp.arange(64 * CHUNK, dtype=jnp.int32) % 72])
    parts0 = _sc_aggregate(60, 3072, 3072, filter_below=N2)(
        tbl0, src0, dst0, cinit0)
    h_aug = _tc_dense(parts0, x[:N2], W_l0, b_l0.reshape(1, D), W_r0,
                      relu=True, aug=True)

    src1, dst1 = _pad_edges(edge_index_1, 24, N2)
    parts1 = _sc_aggregate(24, 3072, 3072)(h_aug, src1, dst1)
    h2 = _tc_dense(parts1, h_aug[:, :D], W_l1, b_l1.reshape(1, D), W_r1,
                   relu=False, aug=False)

    third = N2 // 3
    return (h2[:third], h2[third:2 * third], h2[2 * third:])


# speedup vs baseline: 1.0947x; 1.0947x over previous
import functools

import jax
import jax.numpy as jnp
from jax import lax
from jax.experimental import pallas as pl
from jax.experimental.pallas import tpu as pltpu
from jax.experimental.pallas import tpu_sc as plsc

N0, N1, N2 = 10000, 7500, 3000
D = 128
DA = 144
NC, NS = 2, 16
NW = NC * NS
CHUNK = 128


def _sc_aggregate(n_chunks, acc_rows, out_rows, filter_below=None):
    assert n_chunks % 2 == 0
    crows = n_chunks + 4
    zrows = acc_rows // NS
    orows = out_rows // NS

    mesh = plsc.VectorSubcoreMesh(
        core_axis_name="c", subcore_axis_name="s",
        num_cores=NC, num_subcores=NS)

    scratch = [
        pltpu.VMEM_SHARED((acc_rows, DA), jnp.float32),
        pltpu.VMEM((n_chunks, CHUNK), jnp.int32),
        pltpu.VMEM((n_chunks, CHUNK), jnp.int32),
        pltpu.VMEM((CHUNK, DA), jnp.float32),
        pltpu.VMEM((CHUNK, DA), jnp.float32),
        pltpu.VMEM((16, DA), jnp.float32),
        pltpu.SemaphoreType.DMA,
        pltpu.SemaphoreType.DMA,
    ]
    if filter_below is not None:
        scratch += [
            pltpu.VMEM((crows * CHUNK,), jnp.int32),
            pltpu.VMEM((crows * CHUNK,), jnp.int32),
        ]

    @functools.partial(
        pl.kernel,
        out_type=jax.ShapeDtypeStruct((NC, out_rows, DA), jnp.float32),
        mesh=mesh,
        scratch_types=scratch,
        compiler_params=pltpu.CompilerParams(use_tc_tiling_on_sc=False, needs_layout_passes=False),
    )
    def agg(table, srcs, dsts, *rest):
        if filter_below is None:
            (parts, acc, src_v, dst_v, rows0, rows1, zbuf,
             sem0, sem1) = rest
        else:
            (cinit, parts, acc, src_v, dst_v, rows0, rows1, zbuf,
             sem0, sem1, csrc, cdst) = rest
        c = lax.axis_index("c")
        s = lax.axis_index("s")
        wid = c * NS + s

        for i in range(16):
            for j in range(DA // 16):
                zbuf[i, pl.ds(j * 16, 16)] = jnp.zeros((16,), jnp.float32)

        def zero_body(k, _):
            pltpu.sync_copy(zbuf, acc.at[pl.ds(s * zrows + k * 16, 16)])
            return 0
        lax.fori_loop(0, zrows // 16, zero_body, 0)

        pltpu.sync_copy(srcs.at[wid], src_v)
        pltpu.sync_copy(dsts.at[wid], dst_v)
        plsc.subcore_barrier()

        if filter_below is None:
            gsrc, gdst = src_v, dst_v
            n_loop = n_chunks
        else:
            pltpu.sync_copy(cinit.at[0], csrc)
            pltpu.sync_copy(cinit.at[1], cdst)

            def cmp_body(j, off):
                for t in range(CHUNK // 16):
                    sv = src_v[j, pl.ds(t * 16, 16)]
                    dv = dst_v[j, pl.ds(t * 16, 16)]
                    m = dv < filter_below
                    plsc.store_compressed(csrc.at[pl.ds(off, 16)], sv, mask=m)
                    plsc.store_compressed(cdst.at[pl.ds(off, 16)], dv, mask=m)
                    off = off + plsc.all_reduce_population_count(m)[0]
                return off
            n_kept = lax.fori_loop(0, n_chunks, cmp_body,
                                   jnp.int32(0))
            gsrc, gdst = csrc, cdst
            n_loop = jnp.right_shift(n_kept + CHUNK - 1, 7)

        if filter_below is None:
            def edge_body(j, _):
                pltpu.async_copy(table.at[gsrc.at[j]], rows0, sem0).wait()
                pltpu.sync_copy(rows0, acc.at[gdst.at[j]], add=True)
                return 0
        else:
            def edge_body(j, _):
                sl = pl.ds(j * CHUNK, CHUNK)
                pltpu.async_copy(table.at[gsrc.at[sl]], rows0, sem0).wait()
                pltpu.sync_copy(rows0, acc.at[gdst.at[sl]], add=True)
                return 0
        lax.fori_loop(0, n_loop, edge_body, 0)

        plsc.subcore_barrier()
        pltpu.sync_copy(acc.at[pl.ds(s * orows, orows)],
                        parts.at[c, pl.ds(s * orows, orows)])

    return agg


def _tc_dense(parts, x_tgt, W_l, b_l, W_r, relu, aug):
    n = x_tgt.shape[0]
    out_w = DA if aug else D

    def body(p_ref, x_ref, wl_ref, bl_ref, wr_ref, o_ref):
        acc = p_ref[0] + p_ref[1]
        feat = acc[:n, :D]
        cnt = acc[:n, D:D + 1]
        mean = feat / jnp.maximum(cnt, 1.0)
        h = (jnp.dot(mean, wl_ref[...], preferred_element_type=jnp.float32)
             + bl_ref[...]
             + jnp.dot(x_ref[...], wr_ref[...],
                       preferred_element_type=jnp.float32))
        if relu:
            h = jnp.maximum(h, 0.0)
        if aug:
            o_ref[:, :D] = h
            lane = lax.broadcasted_iota(jnp.int32, (n, DA - D), 1)
            o_ref[:, D:] = jnp.where(lane == 0, 1.0, 0.0)
        else:
            o_ref[...] = h

    return pl.pallas_call(
        body,
        out_shape=jax.ShapeDtypeStruct((n, out_w), jnp.float32),
    )(parts, x_tgt, W_l, b_l, W_r)


def _pad_edges(edge_index, n_chunks, dummy, spread=72):
    e = edge_index.shape[1]
    assert e % NW == 0
    per, slots = e // NW, n_chunks * CHUNK
    src = jnp.concatenate(
        [edge_index[0].astype(jnp.int32).reshape(NW, per),
         jnp.zeros((NW, slots - per), jnp.int32)], axis=1)
    dst = jnp.concatenate(
        [edge_index[1].astype(jnp.int32).reshape(NW, per),
         dummy + jnp.broadcast_to(
             jnp.arange(slots - per, dtype=jnp.int32) % spread,
             (NW, slots - per))], axis=1)
    return (src.reshape(NW, n_chunks, CHUNK),
            dst.reshape(NW, n_chunks, CHUNK))


def kernel(x, edge_index_0, edge_index_1, W_l0, b_l0, W_r0, W_l1, b_l1, W_r1):
    tbl0 = jnp.concatenate(
        [x[:N1], jnp.ones((N1, 1), jnp.float32),
         jnp.zeros((N1, DA - D - 1), jnp.float32)], axis=1)
    src0, dst0 = _pad_edges(edge_index_0, 60, N1)
    cinit0 = jnp.stack([jnp.zeros((64 * CHUNK,), jnp.int32),
                        N2 + jnp.arange(64 * CHUNK, dtype=jnp.int32) % 72])
    parts0 = _sc_aggregate(60, 3072, 3072, filter_below=N2)(
        tbl0, src0, dst0, cinit0)
    h_aug = _tc_dense(parts0, x[:N2], W_l0, b_l0.reshape(1, D), W_r0,
                      relu=True, aug=True)

    src1, dst1 = _pad_edges(edge_index_1, 24, N2)
    parts1 = _sc_aggregate(24, 3072, 3072)(h_aug, src1, dst1)
    h2 = _tc_dense(parts1, h_aug[:, :D], W_l1, b_l1.reshape(1, D), W_r1,
                   relu=False, aug=False)

    third = N2 // 3
    return (h2[:third], h2[third:2 * third], h2[2 * third:])


# speedup vs baseline: 1.3978x; 1.2769x over previous
import functools

import jax
import jax.numpy as jnp
from jax import lax
from jax.experimental import pallas as pl
from jax.experimental.pallas import tpu as pltpu
from jax.experimental.pallas import tpu_sc as plsc

N0, N1, N2 = 10000, 7500, 3000
D = 128
DA = 144
NC, NS = 2, 16
NW = NC * NS
CHUNK = 128


def _sc_aggregate(n_chunks, acc_rows, out_rows, filter_below=None,
                  stage_rows=None):
    assert n_chunks % 2 == 0
    crows = n_chunks + 4
    zrows = acc_rows // NS
    orows = out_rows // NS

    mesh = plsc.VectorSubcoreMesh(
        core_axis_name="c", subcore_axis_name="s",
        num_cores=NC, num_subcores=NS)

    scratch = [
        pltpu.VMEM_SHARED((acc_rows, DA), jnp.float32),
        pltpu.VMEM((n_chunks, CHUNK), jnp.int32),
        pltpu.VMEM((n_chunks, CHUNK), jnp.int32),
        pltpu.VMEM((CHUNK, DA), jnp.float32),
        pltpu.VMEM((CHUNK, DA), jnp.float32),
        pltpu.VMEM((16, DA), jnp.float32),
        pltpu.SemaphoreType.DMA,
        pltpu.SemaphoreType.DMA,
    ]
    if filter_below is not None:
        scratch += [
            pltpu.VMEM((crows * CHUNK,), jnp.int32),
            pltpu.VMEM((crows * CHUNK,), jnp.int32),
        ]
    if stage_rows is not None:
        scratch += [pltpu.VMEM_SHARED((stage_rows, DA), jnp.float32)]

    @functools.partial(
        pl.kernel,
        out_type=jax.ShapeDtypeStruct((NC, out_rows, DA), jnp.float32),
        mesh=mesh,
        scratch_types=scratch,
        compiler_params=pltpu.CompilerParams(use_tc_tiling_on_sc=False, needs_layout_passes=False),
    )
    def agg(table, srcs, dsts, *rest):
        if filter_below is None:
            (parts, acc, src_v, dst_v, rows0, rows1, zbuf,
             sem0, sem1) = rest[:9]
        else:
            (cinit, parts, acc, src_v, dst_v, rows0, rows1, zbuf,
             sem0, sem1, csrc, cdst) = rest[:12]
        tbl_sh = rest[-1] if stage_rows is not None else None
        c = lax.axis_index("c")
        s = lax.axis_index("s")
        wid = c * NS + s

        for i in range(16):
            for j in range(DA // 16):
                zbuf[i, pl.ds(j * 16, 16)] = jnp.zeros((16,), jnp.float32)

        def zero_body(k, _):
            pltpu.sync_copy(zbuf, acc.at[pl.ds(s * zrows + k * 16, 16)])
            return 0
        lax.fori_loop(0, zrows // 16, zero_body, 0)

        pltpu.sync_copy(srcs.at[wid], src_v)
        pltpu.sync_copy(dsts.at[wid], dst_v)
        if stage_rows is not None:
            trows = stage_rows // NS
            pltpu.sync_copy(table.at[pl.ds(s * trows, trows)],
                            tbl_sh.at[pl.ds(s * trows, trows)])
        plsc.subcore_barrier()
        gtab = table if stage_rows is None else tbl_sh

        if filter_below is None:
            gsrc, gdst = src_v, dst_v
            n_loop = n_chunks
        else:
            pltpu.sync_copy(cinit.at[0], csrc)
            pltpu.sync_copy(cinit.at[1], cdst)

            def cmp_body(j, off):
                for t in range(CHUNK // 16):
                    sv = src_v[j, pl.ds(t * 16, 16)]
                    dv = dst_v[j, pl.ds(t * 16, 16)]
                    m = dv < filter_below
                    plsc.store_compressed(csrc.at[pl.ds(off, 16)], sv, mask=m)
                    plsc.store_compressed(cdst.at[pl.ds(off, 16)], dv, mask=m)
                    off = off + plsc.all_reduce_population_count(m)[0]
                return off
            n_kept = lax.fori_loop(0, n_chunks, cmp_body,
                                   jnp.int32(0))
            gsrc, gdst = csrc, cdst
            n_loop = jnp.right_shift(n_kept + CHUNK - 1, 7)

        if filter_below is None:
            def edge_body(j, _):
                pltpu.async_copy(gtab.at[gsrc.at[j]], rows0, sem0).wait()
                pltpu.sync_copy(rows0, acc.at[gdst.at[j]], add=True)
                return 0
        else:
            def edge_body(j, _):
                sl = pl.ds(j * CHUNK, CHUNK)
                pltpu.async_copy(gtab.at[gsrc.at[sl]], rows0, sem0).wait()
                pltpu.sync_copy(rows0, acc.at[gdst.at[sl]], add=True)
                return 0
        lax.fori_loop(0, n_loop, edge_body, 0)

        plsc.subcore_barrier()
        pltpu.sync_copy(acc.at[pl.ds(s * orows, orows)],
                        parts.at[c, pl.ds(s * orows, orows)])

    return agg


def _tc_dense(parts, x_tgt, W_l, b_l, W_r, relu, aug):
    n = x_tgt.shape[0]
    out_w = DA if aug else D

    def body(p_ref, x_ref, wl_ref, bl_ref, wr_ref, o_ref):
        acc = p_ref[0] + p_ref[1]
        feat = acc[:n, :D]
        cnt = acc[:n, D:D + 1]
        mean = feat / jnp.maximum(cnt, 1.0)
        h = (jnp.dot(mean, wl_ref[...], preferred_element_type=jnp.float32)
             + bl_ref[...]
             + jnp.dot(x_ref[...], wr_ref[...],
                       preferred_element_type=jnp.float32))
        if relu:
            h = jnp.maximum(h, 0.0)
        if aug:
            o_ref[:, :D] = h
            lane = lax.broadcasted_iota(jnp.int32, (n, DA - D), 1)
            o_ref[:, D:] = jnp.where(lane == 0, 1.0, 0.0)
        else:
            o_ref[...] = h

    return pl.pallas_call(
        body,
        out_shape=jax.ShapeDtypeStruct((n, out_w), jnp.float32),
    )(parts, x_tgt, W_l, b_l, W_r)


def _pad_edges(edge_index, n_chunks, dummy, spread=72):
    e = edge_index.shape[1]
    assert e % NW == 0
    per, slots = e // NW, n_chunks * CHUNK
    src = jnp.concatenate(
        [edge_index[0].astype(jnp.int32).reshape(NW, per),
         jnp.zeros((NW, slots - per), jnp.int32)], axis=1)
    dst = jnp.concatenate(
        [edge_index[1].astype(jnp.int32).reshape(NW, per),
         dummy + jnp.broadcast_to(
             jnp.arange(slots - per, dtype=jnp.int32) % spread,
             (NW, slots - per))], axis=1)
    return (src.reshape(NW, n_chunks, CHUNK),
            dst.reshape(NW, n_chunks, CHUNK))


def kernel(x, edge_index_0, edge_index_1, W_l0, b_l0, W_r0, W_l1, b_l1, W_r1):
    tbl0 = jnp.concatenate(
        [x[:N1], jnp.ones((N1, 1), jnp.float32),
         jnp.zeros((N1, DA - D - 1), jnp.float32)], axis=1)
    src0, dst0 = _pad_edges(edge_index_0, 60, N1)
    cinit0 = jnp.stack([jnp.zeros((64 * CHUNK,), jnp.int32),
                        N2 + jnp.arange(64 * CHUNK, dtype=jnp.int32) % 72])
    parts0 = _sc_aggregate(60, 3072, 3072, filter_below=N2)(
        tbl0, src0, dst0, cinit0)
    h_aug = _tc_dense(parts0, x[:N2], W_l0, b_l0.reshape(1, D), W_r0,
                      relu=True, aug=True)

    src1, dst1 = _pad_edges(edge_index_1, 24, N2)
    h_pad = jnp.concatenate(
        [h_aug, jnp.zeros((3072 - N2, DA), jnp.float32)], axis=0)
    parts1 = _sc_aggregate(24, 3072, 3072, stage_rows=3072)(
        h_pad, src1, dst1)
    h2 = _tc_dense(parts1, h_aug[:, :D], W_l1, b_l1.reshape(1, D), W_r1,
                   relu=False, aug=False)

    third = N2 // 3
    return (h2[:third], h2[third:2 * third], h2[2 * third:])
